# Initial kernel scaffold; baseline (speedup 1.0000x reference)
import jax
import jax.numpy as jnp
from jax.experimental import pallas as pl


def kernel(tokens, table):
    raise NotImplementedError("write your pallas kernel here")


# speedup vs baseline: 1.3998x; 1.3998x over previous
import functools
import math

import jax
import jax.numpy as jnp
from jax import lax
from jax.experimental import pallas as pl
from jax.experimental.pallas import tpu as pltpu
from jax.experimental.pallas import tpu_sc as plsc


def _emb_body(tok_hbm, tab_hbm, out_hbm, idx_v, rows_v, gsem,
              *, n_chunks, chunk, b_per_w, d, lanes, nc):
    scale = jnp.float32(math.sqrt(d))
    wid = lax.axis_index("s") * nc + lax.axis_index("c")
    base = wid * b_per_w

    def chunk_body(c, carry):
        off = base + c * chunk
        pltpu.sync_copy(tok_hbm.at[pl.ds(off, chunk)], idx_v.at[0])
        pltpu.async_copy(tab_hbm.at[idx_v.at[0]], rows_v.at[0], gsem).wait()

        def row_body(j, carry2):
            for h in range(d // lanes):
                sl = pl.ds(h * lanes, lanes)
                rows_v[0, j, sl] = rows_v[0, j, sl] * scale
            return carry2

        lax.fori_loop(0, chunk, row_body, 0, unroll=4)
        pltpu.sync_copy(rows_v.at[0], out_hbm.at[pl.ds(off, chunk)])
        return carry

    lax.fori_loop(0, n_chunks, chunk_body, 0)


def kernel(tokens, table):
    b0, hist = tokens.shape
    v, d = table.shape
    b = b0 * hist
    flat = tokens.reshape(b)

    info = plsc.get_sparse_core_info()
    nc, ns, lanes = info.num_cores, info.num_subcores, info.num_lanes
    nw = nc * ns
    b_per_w = b // nw
    chunk = 1024
    n_chunks = b_per_w // chunk
    assert b % nw == 0 and b_per_w % chunk == 0 and d % lanes == 0

    mesh = plsc.VectorSubcoreMesh(core_axis_name="c", subcore_axis_name="s")
    body = functools.partial(
        _emb_body, n_chunks=n_chunks, chunk=chunk, b_per_w=b_per_w,
        d=d, lanes=lanes, nc=nc)

    k = functools.partial(
        pl.kernel,
        mesh=mesh,
        compiler_params=pltpu.CompilerParams(use_tc_tiling_on_sc=False),
        out_type=jax.ShapeDtypeStruct((b, d), jnp.float32),
        scratch_types=[
            pltpu.VMEM((1, chunk), jnp.int32),
            pltpu.VMEM((1, chunk, d), jnp.float32),
            pltpu.SemaphoreType.DMA,
        ],
    )(body)

    out = k(flat, table)
    return out.reshape(b0, hist, d)


# speedup vs baseline: 1.4768x; 1.0549x over previous
import functools
import math

import jax
import jax.numpy as jnp
from jax import lax
from jax.experimental import pallas as pl
from jax.experimental.pallas import tpu as pltpu
from jax.experimental.pallas import tpu_sc as plsc

_NBUF = 4
_INFLIGHT = 2


def _emb_body(tok_hbm, tab_hbm, out_hbm, idx_v, rows_v, *sems,
              n_chunks, chunk, b_per_w, d, lanes, nc):
    gsems = sems[:_NBUF]
    wsems = sems[_NBUF:]
    scale = jnp.float32(math.sqrt(d))
    wid = lax.axis_index("s") * nc + lax.axis_index("c")
    base = wid * b_per_w

    def issue_gather(c, s):
        pltpu.sync_copy(tok_hbm.at[pl.ds(base + c * chunk, chunk)],
                        idx_v.at[s])
        pltpu.make_async_copy(tab_hbm.at[idx_v.at[s]], rows_v.at[s],
                              gsems[s]).start()

    for c0 in range(_INFLIGHT):
        issue_gather(c0, c0)

    n_outer = n_chunks // _NBUF

    def outer(o, carry):
        for s in range(_NBUF):
            c = o * _NBUF + s
            pltpu.make_async_copy(tab_hbm.at[idx_v.at[s]], rows_v.at[s],
                                  gsems[s]).wait()

            def row_body(j, carry2):
                for h in range(d // lanes):
                    sl = pl.ds(h * lanes, lanes)
                    rows_v[s, j, sl] = rows_v[s, j, sl] * scale
                return carry2

            lax.fori_loop(0, chunk, row_body, 0, unroll=4)

            pltpu.make_async_copy(rows_v.at[s],
                                  out_hbm.at[pl.ds(base + c * chunk, chunk)],
                                  wsems[s]).start()

            cn = c + _INFLIGHT
            sn = (s + _INFLIGHT) % _NBUF

            @pl.when(cn < n_chunks)
            def _():
                @pl.when(cn >= _NBUF)
                def _():
                    pltpu.make_async_copy(
                        rows_v.at[sn], out_hbm.at[pl.ds(base, chunk)],
                        wsems[sn]).wait()
                issue_gather(cn, sn)
        return carry

    lax.fori_loop(0, n_outer, outer, 0)

    for s in range(_NBUF):
        pltpu.make_async_copy(rows_v.at[s], out_hbm.at[pl.ds(base, chunk)],
                              wsems[s]).wait()


def kernel(tokens, table):
    b0, hist = tokens.shape
    v, d = table.shape
    b = b0 * hist
    flat = tokens.reshape(b)

    info = plsc.get_sparse_core_info()
    nc, ns, lanes = info.num_cores, info.num_subcores, info.num_lanes
    nw = nc * ns
    b_per_w = b // nw
    chunk = 800
    n_chunks = b_per_w // chunk
    assert b % nw == 0 and b_per_w % chunk == 0 and d % lanes == 0
    assert n_chunks % _NBUF == 0 and chunk % 8 == 0

    mesh = plsc.VectorSubcoreMesh(core_axis_name="c", subcore_axis_name="s")
    body = functools.partial(
        _emb_body, n_chunks=n_chunks, chunk=chunk, b_per_w=b_per_w,
        d=d, lanes=lanes, nc=nc)

    k = functools.partial(
        pl.kernel,
        mesh=mesh,
        compiler_params=pltpu.CompilerParams(use_tc_tiling_on_sc=False),
        out_type=jax.ShapeDtypeStruct((b, d), jnp.float32),
        scratch_types=[
            pltpu.VMEM((_NBUF, chunk), jnp.int32),
            pltpu.VMEM((_NBUF, chunk, d), jnp.float32),
        ] + [pltpu.SemaphoreType.DMA] * (2 * _NBUF),
    )(body)

    out = k(flat, table)
    return out.reshape(b0, hist, d)
